# baseline (speedup 1.0000x reference)
import functools

import jax
import jax.numpy as jnp
import numpy as np
from jax import lax
from jax.experimental import pallas as pl
from jax.experimental.pallas import tpu as pltpu
from jax.experimental.pallas import tpu_sc as plsc

L = 16
SUB = 16
PI = np.float32(np.pi)
PI4 = np.float32(np.pi / 4)
HALF = np.float32(0.5)
ONE = np.float32(1.0)
ZERO = np.float32(0.0)
EPS = np.float32(1e-8)
THR = np.float32(0.25)


def _iou_vs(tx1, ty1, tx2, ty2, ta, x1, y1, x2, y2, a):
    ix1 = jnp.maximum(tx1, x1)
    iy1 = jnp.maximum(ty1, y1)
    ix2 = jnp.minimum(tx2, x2)
    iy2 = jnp.minimum(ty2, y2)
    w = jnp.maximum(ix2 - ix1, ZERO)
    h = jnp.maximum(iy2 - iy1, ZERO)
    inter = w * h
    denom = ta + a - inter + EPS
    return inter / denom


def _nms_body(npad, nreal, boxes_t, order_h, scores_h, out,
              c0, c1, c2, c3, c4, c5, c6, ov, sv,
              bx1, by1, bx2, by2,
              ax1, ay1, ax2, ay2, aa, apos, amask, kp, ko):
    nb = npad // L
    cid = lax.axis_index("c")
    sid = lax.axis_index("s")
    iota = lax.iota(jnp.int32, L)

    def _lane(vec, t):
        return jnp.sum(jnp.where(iota == t, vec, ZERO))

    @pl.when(jnp.logical_and(cid == 0, sid == 0))
    def _main():
        cols = [c0, c1, c2, c3, c4, c5, c6]
        for k in range(7):
            pltpu.sync_copy(boxes_t.at[k], cols[k])
        pltpu.sync_copy(order_h, ov)
        pltpu.sync_copy(scores_h, sv)

        def bev_body(j, _):
            sl = pl.ds(j * L, L)
            x = c0[sl]
            y = c1[sl]
            w = c3[sl]
            ll = c4[sl]
            r = c6[sl]
            t = r / PI + HALF
            tr = t.astype(jnp.int32).astype(jnp.float32)
            fl = tr - jnp.where(t < tr, ONE, ZERO)
            ang = r - fl * PI
            cond = jnp.abs(ang) > PI4
            we = jnp.where(cond, ll, w)
            le = jnp.where(cond, w, ll)
            bx1[sl] = x - we * HALF
            by1[sl] = y - le * HALF
            bx2[sl] = x + we * HALF
            by2[sl] = y + le * HALF
            return 0

        lax.fori_loop(0, nb, bev_body, 0)

        def gather_body(j, _):
            sl = pl.ds(j * L, L)
            idx = ov[sl]
            gx1 = plsc.load_gather(bx1, [idx])
            gy1 = plsc.load_gather(by1, [idx])
            gx2 = plsc.load_gather(bx2, [idx])
            gy2 = plsc.load_gather(by2, [idx])
            ax1[sl] = gx1
            ay1[sl] = gy1
            ax2[sl] = gx2
            ay2[sl] = gy2
            aa[sl] = (gx2 - gx1) * (gy2 - gy1)
            apos[sl] = j * L + iota
            kp[sl] = jnp.broadcast_to(ZERO, (L,))
            return 0

        lax.fori_loop(0, nb, gather_body, 0)

        def round_body(carry):
            count, rnd = carry
            bsl = pl.ds(0, L)
            x1b = ax1[bsl]
            y1b = ay1[bsl]
            x2b = ax2[bsl]
            y2b = ay2[bsl]
            ab = aa[bsl]
            posb = apos[bsl]
            valid = iota < count
            kv2 = jnp.where(valid, ONE, ZERO)
            for t in range(L):
                iou = _iou_vs(_lane(x1b, t), _lane(y1b, t), _lane(x2b, t),
                              _lane(y2b, t), _lane(ab, t),
                              x1b, y1b, x2b, y2b, ab)
                supp = jnp.logical_and(iou > THR, iota > t)
                kt = jnp.max(jnp.where(iota == t, kv2, ZERO))
                kv2 = kv2 * (ONE - jnp.where(supp, ONE, ZERO) * kt)
            plsc.store_scatter(kp, [posb], kv2, mask=valid)

            kts = [jnp.max(jnp.where(iota == t, kv2, ZERO)) for t in range(L)]

            nchunks = (count - 1) // L

            npass = L // SUB
            for p in range(npass):
                ts = range(p * SUB, (p + 1) * SUB)
                last = p == npass - 1

                def make_splats():
                    return [(_lane(x1b, t), _lane(y1b, t), _lane(x2b, t),
                             _lane(y2b, t), _lane(ab, t), kts[t])
                            for t in ts]

                if not last:
                    ksum = functools.reduce(lambda a, b: a + b,
                                            [kts[t] for t in ts])

                    @pl.when(ksum > HALF)
                    def _pass():
                        spl = make_splats()

                        def pass_body(j, _):
                            sl = pl.ds(L + j * L, L)
                            x1 = ax1[sl]
                            y1 = ay1[sl]
                            x2 = ax2[sl]
                            y2 = ay2[sl]
                            a = aa[sl]
                            f = amask[pl.ds(j * L, L)] if p else jnp.where(
                                L + j * L + iota < count, ONE, ZERO)
                            for (tx1, ty1, tx2, ty2, ta, ktv) in spl:
                                iou = _iou_vs(tx1, ty1, tx2, ty2, ta,
                                              x1, y1, x2, y2, a)
                                f = f * (ONE - jnp.where(iou > THR, ONE,
                                                         ZERO) * ktv)
                            amask[pl.ds(j * L, L)] = f
                            return 0

                        lax.fori_loop(0, nchunks, pass_body, 0)

                    if p == 0:
                        @pl.when(jnp.logical_not(ksum > HALF))
                        def _init():
                            def init_body(j, _):
                                amask[pl.ds(j * L, L)] = jnp.where(
                                    L + j * L + iota < count, ONE, ZERO)
                                return 0

                            lax.fori_loop(0, nchunks, init_body, 0)
                else:
                    spl = make_splats()

                    def compact_body(j, off):
                        sl = pl.ds(L + j * L, L)
                        x1 = ax1[sl]
                        y1 = ay1[sl]
                        x2 = ax2[sl]
                        y2 = ay2[sl]
                        a = aa[sl]
                        pos = apos[sl]
                        f = (amask[pl.ds(j * L, L)] if npass > 1 else
                             jnp.where(L + j * L + iota < count, ONE, ZERO))
                        for (tx1, ty1, tx2, ty2, ta, ktv) in spl:
                            iou = _iou_vs(tx1, ty1, tx2, ty2, ta,
                                          x1, y1, x2, y2, a)
                            f = f * (ONE - jnp.where(iou > THR, ONE,
                                                     ZERO) * ktv)
                        m = f > HALF
                        mi = jnp.where(m, 1, 0).astype(jnp.int32)
                        dst = off + plsc.cumsum(mi) - mi
                        plsc.store_scatter(ax1, [dst], x1, mask=m)
                        plsc.store_scatter(ay1, [dst], y1, mask=m)
                        plsc.store_scatter(ax2, [dst], x2, mask=m)
                        plsc.store_scatter(ay2, [dst], y2, mask=m)
                        plsc.store_scatter(aa, [dst], a, mask=m)
                        plsc.store_scatter(apos, [dst], pos, mask=m)
                        cnt = jnp.max(plsc.all_reduce_population_count(m))
                        return off + cnt

                    new_count = lax.fori_loop(0, nchunks, compact_body,
                                              jnp.int32(0))

            return (new_count, rnd + 1)

        lax.while_loop(lambda c: c[0] > 0, round_body,
                       (jnp.int32(nreal), jnp.int32(0)))

        def scatter_body(j, _):
            sl = pl.ds(j * L, L)
            plsc.store_scatter(ko, [ov[sl]], kp[sl])
            return 0

        lax.fori_loop(0, nb, scatter_body, 0)

        def mask_body(j, _):
            sl = pl.ds(j * L, L)
            k = ko[sl]
            for ref in (c0, c1, c2, c3, c4, c5, c6, sv):
                ref[sl] = ref[sl] * k
            return 0

        lax.fori_loop(0, nb, mask_body, 0)
        for k in range(7):
            pltpu.sync_copy(cols[k], out.at[k])
        pltpu.sync_copy(sv, out.at[7])


@functools.partial(jax.jit, static_argnames=("npad", "nreal"))
def _nms_sc(boxes_t, order_p, scores_p, *, npad, nreal):
    f32 = jnp.float32
    i32 = jnp.int32
    scratch = (
        [pltpu.VMEM((npad,), f32) for _ in range(7)]
        + [pltpu.VMEM((npad,), i32)]
        + [pltpu.VMEM((npad,), f32)]
        + [pltpu.VMEM((npad,), f32) for _ in range(4)]
        + [pltpu.VMEM((npad + L,), f32) for _ in range(5)]
        + [pltpu.VMEM((npad + L,), i32)]
        + [pltpu.VMEM((npad,), f32)]
        + [pltpu.VMEM((npad,), f32) for _ in range(2)]
    )
    mesh = plsc.VectorSubcoreMesh(core_axis_name="c", subcore_axis_name="s",
                                  num_cores=2, num_subcores=16)
    return pl.kernel(
        functools.partial(_nms_body, npad, nreal),
        out_type=jax.ShapeDtypeStruct((8, npad), f32),
        mesh=mesh,
        scratch_types=scratch,
        compiler_params=pltpu.CompilerParams(use_tc_tiling_on_sc=False,
                                             needs_layout_passes=False),
    )(boxes_t, order_p, scores_p)


def kernel(boxes, scores):
    n = boxes.shape[0]
    npad = ((n + L - 1) // L) * L
    order = jnp.argsort(-scores).astype(jnp.int32)
    order_p = jnp.concatenate(
        [order, jnp.arange(n, npad, dtype=jnp.int32)])
    boxes_t = jnp.pad(boxes, ((0, npad - n), (0, 0))).T
    scores_p = jnp.pad(scores, (0, npad - n))
    out_t = _nms_sc(boxes_t, order_p, scores_p, npad=npad, nreal=n)
    return out_t[:, :n].T


# speedup vs baseline: 1.2157x; 1.2157x over previous
import functools

import jax
import jax.numpy as jnp
import numpy as np
from jax import lax
from jax.experimental import pallas as pl
from jax.experimental.pallas import tpu as pltpu
from jax.experimental.pallas import tpu_sc as plsc

L = 16
SUB = 8
PI = np.float32(np.pi)
PI4 = np.float32(np.pi / 4)
HALF = np.float32(0.5)
ONE = np.float32(1.0)
ZERO = np.float32(0.0)
EPS = np.float32(1e-8)
THR = np.float32(0.25)


def _iou_vs(tx1, ty1, tx2, ty2, ta, x1, y1, x2, y2, a):
    ix1 = jnp.maximum(tx1, x1)
    iy1 = jnp.maximum(ty1, y1)
    ix2 = jnp.minimum(tx2, x2)
    iy2 = jnp.minimum(ty2, y2)
    w = jnp.maximum(ix2 - ix1, ZERO)
    h = jnp.maximum(iy2 - iy1, ZERO)
    inter = w * h
    denom = ta + a - inter + EPS
    return inter / denom


def _nms_body(npad, nreal, boxes_t, order_h, scores_h, out,
              c0, c1, c2, c3, c4, c5, c6, ov, sv,
              bx1, by1, bx2, by2,
              ax1, ay1, ax2, ay2, aa, apos, amask, kp, ko):
    nb = npad // L
    cid = lax.axis_index("c")
    sid = lax.axis_index("s")
    iota = lax.iota(jnp.int32, L)

    def _lane(vec, t):
        return jnp.sum(jnp.where(iota == t, vec, ZERO))

    @pl.when(jnp.logical_and(cid == 0, sid == 0))
    def _main():
        cols = [c0, c1, c2, c3, c4, c5, c6]
        for k in range(7):
            pltpu.sync_copy(boxes_t.at[k], cols[k])
        pltpu.sync_copy(order_h, ov)
        pltpu.sync_copy(scores_h, sv)

        def bev_body(j, _):
            sl = pl.ds(j * L, L)
            x = c0[sl]
            y = c1[sl]
            w = c3[sl]
            ll = c4[sl]
            r = c6[sl]
            t = r / PI + HALF
            tr = t.astype(jnp.int32).astype(jnp.float32)
            fl = tr - jnp.where(t < tr, ONE, ZERO)
            ang = r - fl * PI
            cond = jnp.abs(ang) > PI4
            we = jnp.where(cond, ll, w)
            le = jnp.where(cond, w, ll)
            bx1[sl] = x - we * HALF
            by1[sl] = y - le * HALF
            bx2[sl] = x + we * HALF
            by2[sl] = y + le * HALF
            return 0

        lax.fori_loop(0, nb, bev_body, 0)

        def gather_body(j, _):
            sl = pl.ds(j * L, L)
            idx = ov[sl]
            gx1 = plsc.load_gather(bx1, [idx])
            gy1 = plsc.load_gather(by1, [idx])
            gx2 = plsc.load_gather(bx2, [idx])
            gy2 = plsc.load_gather(by2, [idx])
            ax1[sl] = gx1
            ay1[sl] = gy1
            ax2[sl] = gx2
            ay2[sl] = gy2
            aa[sl] = (gx2 - gx1) * (gy2 - gy1)
            apos[sl] = j * L + iota
            kp[sl] = jnp.broadcast_to(ZERO, (L,))
            return 0

        lax.fori_loop(0, nb, gather_body, 0)

        def round_body(carry):
            count, rnd = carry
            bsl = pl.ds(0, L)
            x1b = ax1[bsl]
            y1b = ay1[bsl]
            x2b = ax2[bsl]
            y2b = ay2[bsl]
            ab = aa[bsl]
            posb = apos[bsl]
            valid = iota < count
            kv2 = jnp.where(valid, ONE, ZERO)
            for t in range(L):
                iou = _iou_vs(_lane(x1b, t), _lane(y1b, t), _lane(x2b, t),
                              _lane(y2b, t), _lane(ab, t),
                              x1b, y1b, x2b, y2b, ab)
                supp = jnp.logical_and(iou > THR, iota > t)
                kt = jnp.max(jnp.where(iota == t, kv2, ZERO))
                kv2 = kv2 * (ONE - jnp.where(supp, ONE, ZERO) * kt)
            plsc.store_scatter(kp, [posb], kv2, mask=valid)

            kts = [jnp.max(jnp.where(iota == t, kv2, ZERO)) for t in range(L)]

            nchunks = (count - 1) // L

            npass = L // SUB
            for p in range(npass):
                ts = range(p * SUB, (p + 1) * SUB)
                last = p == npass - 1

                def make_splats():
                    return [(_lane(x1b, t), _lane(y1b, t), _lane(x2b, t),
                             _lane(y2b, t), _lane(ab, t), kts[t])
                            for t in ts]

                if not last:
                    ksum = functools.reduce(lambda a, b: a + b,
                                            [kts[t] for t in ts])

                    @pl.when(ksum > HALF)
                    def _pass():
                        spl = make_splats()

                        def pass_body(j, _):
                            sl = pl.ds(L + j * L, L)
                            x1 = ax1[sl]
                            y1 = ay1[sl]
                            x2 = ax2[sl]
                            y2 = ay2[sl]
                            a = aa[sl]
                            f = amask[pl.ds(j * L, L)] if p else jnp.where(
                                L + j * L + iota < count, ONE, ZERO)
                            for (tx1, ty1, tx2, ty2, ta, ktv) in spl:
                                iou = _iou_vs(tx1, ty1, tx2, ty2, ta,
                                              x1, y1, x2, y2, a)
                                f = f * (ONE - jnp.where(iou > THR, ONE,
                                                         ZERO) * ktv)
                            amask[pl.ds(j * L, L)] = f
                            return 0

                        lax.fori_loop(0, nchunks, pass_body, 0)

                    if p == 0:
                        @pl.when(jnp.logical_not(ksum > HALF))
                        def _init():
                            def init_body(j, _):
                                amask[pl.ds(j * L, L)] = jnp.where(
                                    L + j * L + iota < count, ONE, ZERO)
                                return 0

                            lax.fori_loop(0, nchunks, init_body, 0)
                else:
                    spl = make_splats()

                    def compact_body(j, off):
                        sl = pl.ds(L + j * L, L)
                        x1 = ax1[sl]
                        y1 = ay1[sl]
                        x2 = ax2[sl]
                        y2 = ay2[sl]
                        a = aa[sl]
                        pos = apos[sl]
                        f = (amask[pl.ds(j * L, L)] if npass > 1 else
                             jnp.where(L + j * L + iota < count, ONE, ZERO))
                        for (tx1, ty1, tx2, ty2, ta, ktv) in spl:
                            iou = _iou_vs(tx1, ty1, tx2, ty2, ta,
                                          x1, y1, x2, y2, a)
                            f = f * (ONE - jnp.where(iou > THR, ONE,
                                                     ZERO) * ktv)
                        m = f > HALF
                        mi = jnp.where(m, 1, 0).astype(jnp.int32)
                        dst = off + plsc.cumsum(mi) - mi
                        plsc.store_scatter(ax1, [dst], x1, mask=m)
                        plsc.store_scatter(ay1, [dst], y1, mask=m)
                        plsc.store_scatter(ax2, [dst], x2, mask=m)
                        plsc.store_scatter(ay2, [dst], y2, mask=m)
                        plsc.store_scatter(aa, [dst], a, mask=m)
                        plsc.store_scatter(apos, [dst], pos, mask=m)
                        cnt = jnp.max(plsc.all_reduce_population_count(m))
                        return off + cnt

                    new_count = lax.fori_loop(0, nchunks, compact_body,
                                              jnp.int32(0))

            return (new_count, rnd + 1)

        lax.while_loop(lambda c: c[0] > 0, round_body,
                       (jnp.int32(nreal), jnp.int32(0)))

        def scatter_body(j, _):
            sl = pl.ds(j * L, L)
            plsc.store_scatter(ko, [ov[sl]], kp[sl])
            return 0

        lax.fori_loop(0, nb, scatter_body, 0)

        def mask_body(j, _):
            sl = pl.ds(j * L, L)
            k = ko[sl]
            for ref in (c0, c1, c2, c3, c4, c5, c6, sv):
                ref[sl] = ref[sl] * k
            return 0

        lax.fori_loop(0, nb, mask_body, 0)
        for k in range(7):
            pltpu.sync_copy(cols[k], out.at[k])
        pltpu.sync_copy(sv, out.at[7])


@functools.partial(jax.jit, static_argnames=("npad", "nreal"))
def _nms_sc(boxes_t, order_p, scores_p, *, npad, nreal):
    f32 = jnp.float32
    i32 = jnp.int32
    scratch = (
        [pltpu.VMEM((npad,), f32) for _ in range(7)]
        + [pltpu.VMEM((npad,), i32)]
        + [pltpu.VMEM((npad,), f32)]
        + [pltpu.VMEM((npad,), f32) for _ in range(4)]
        + [pltpu.VMEM((npad + L,), f32) for _ in range(5)]
        + [pltpu.VMEM((npad + L,), i32)]
        + [pltpu.VMEM((npad,), f32)]
        + [pltpu.VMEM((npad,), f32) for _ in range(2)]
    )
    mesh = plsc.VectorSubcoreMesh(core_axis_name="c", subcore_axis_name="s",
                                  num_cores=2, num_subcores=16)
    return pl.kernel(
        functools.partial(_nms_body, npad, nreal),
        out_type=jax.ShapeDtypeStruct((8, npad), f32),
        mesh=mesh,
        scratch_types=scratch,
        compiler_params=pltpu.CompilerParams(use_tc_tiling_on_sc=False,
                                             needs_layout_passes=False),
    )(boxes_t, order_p, scores_p)


def kernel(boxes, scores):
    n = boxes.shape[0]
    npad = ((n + L - 1) // L) * L
    order = jnp.argsort(-scores).astype(jnp.int32)
    order_p = jnp.concatenate(
        [order, jnp.arange(n, npad, dtype=jnp.int32)])
    boxes_t = jnp.pad(boxes, ((0, npad - n), (0, 0))).T
    scores_p = jnp.pad(scores, (0, npad - n))
    out_t = _nms_sc(boxes_t, order_p, scores_p, npad=npad, nreal=n)
    return out_t[:, :n].T
